# baseline (speedup 1.0000x reference)
import functools

import jax
import jax.numpy as jnp
import numpy as np
from jax.experimental import pallas as pl
from jax.experimental.pallas import tpu as pltpu

B, H, E, F = 32, 2048, 8, 4096
BH = 256
K = H // BH
BF2 = F // K

_MASK_VAL = float(np.finfo(np.float32).min)


def _moe_kernel(x2_ref, xc_ref, gw_ref, w1_ref, w3_ref, w2_ref, out_ref,
                h1a_ref, h3a_ref, hid_ref):
    ei = pl.program_id(0)
    k = pl.program_id(1)

    @pl.when((ei == 0) & (k == 0))
    def _zero():
        out_ref[...] = jnp.zeros_like(out_ref)

    @pl.when(ei > 0)
    def _consume():
        out_ref[...] += jnp.dot(hid_ref[k], w2_ref[0].astype(jnp.bfloat16),
                                preferred_element_type=jnp.float32)

    @pl.when(ei < E)
    def _produce():
        xb = xc_ref[k].astype(jnp.bfloat16)
        p1 = jnp.dot(xb, w1_ref[0].astype(jnp.bfloat16),
                     preferred_element_type=jnp.float32)
        p3 = jnp.dot(xb, w3_ref[0].astype(jnp.bfloat16),
                     preferred_element_type=jnp.float32)

        @pl.when(k == 0)
        def _init():
            h1a_ref[...] = p1
            h3a_ref[...] = p3

        @pl.when(k > 0)
        def _acc():
            h1a_ref[...] += p1
            h3a_ref[...] += p3

        @pl.when(k == K - 1)
        def _finish():
            logits = jnp.dot(x2_ref[...], gw_ref[...],
                             preferred_element_type=jnp.float32)
            m0 = jnp.max(logits, axis=1, keepdims=True)
            cond0 = logits == m0
            masked = jnp.where(cond0, _MASK_VAL, logits)
            m1 = jnp.max(masked, axis=1, keepdims=True)
            cond1 = logits == m1
            pre = 1.0 / (1.0 + jnp.exp(m1 - m0))
            w_all = (cond0.astype(jnp.float32) * pre
                     - cond1.astype(jnp.float32) * (pre - 1.0))
            onehot = jax.lax.broadcasted_iota(jnp.int32, (1, E), 1) == ei
            scale = jnp.sum(jnp.where(onehot, w_all, 0.0), axis=1,
                            keepdims=True)

            h1 = h1a_ref[...]
            hidden = (h1 * jax.nn.sigmoid(h1)) * h3a_ref[...] * scale
            for kk in range(K):
                hid_ref[kk] = hidden[:, kk * BF2:(kk + 1) * BF2].astype(
                    jnp.bfloat16)


@functools.partial(jax.jit, static_argnames=("interpret",))
def _moe(x, gate_w, w1, w3, w2, interpret=False):
    xc = x.reshape(B, K, BH).transpose(1, 0, 2)
    return pl.pallas_call(
        _moe_kernel,
        grid=(E + 1, K),
        in_specs=[
            pl.BlockSpec((B, H), lambda ei, k: (0, 0)),
            pl.BlockSpec((K, B, BH), lambda ei, k: (0, 0, 0)),
            pl.BlockSpec((H, E), lambda ei, k: (0, 0)),
            pl.BlockSpec((1, BH, F),
                         lambda ei, k: (jnp.minimum(ei, E - 1),
                                        jnp.where(ei == E, K - 1, k), 0)),
            pl.BlockSpec((1, BH, F),
                         lambda ei, k: (jnp.minimum(ei, E - 1),
                                        jnp.where(ei == E, K - 1, k), 0)),
            pl.BlockSpec((1, BF2, H),
                         lambda ei, k: (jnp.maximum(ei - 1, 0),
                                        jnp.where(ei == 0, 0, k), 0)),
        ],
        out_specs=pl.BlockSpec((B, H), lambda ei, k: (0, 0)),
        out_shape=jax.ShapeDtypeStruct((B, H), jnp.float32),
        scratch_shapes=[
            pltpu.VMEM((B, F), jnp.float32),
            pltpu.VMEM((B, F), jnp.float32),
            pltpu.VMEM((K, B, BF2), jnp.bfloat16),
        ],
        compiler_params=pltpu.CompilerParams(
            dimension_semantics=("arbitrary", "arbitrary"),
        ),
        interpret=interpret,
    )(x, xc, gate_w, w1, w3, w2)


def kernel(inputs, gate_w, w1, w3, w2):
    x = inputs.reshape(B, H)
    out = _moe(x, gate_w, w1, w3, w2)
    return out.reshape(1, 1, B, H)


# speedup vs baseline: 1.0240x; 1.0240x over previous
import functools

import jax
import jax.numpy as jnp
import numpy as np
from jax.experimental import pallas as pl
from jax.experimental.pallas import tpu as pltpu

B, H, E, F = 32, 2048, 8, 4096
BF = 512
NF = F // BF

_MASK_VAL = float(np.finfo(np.float32).min)


def _moe_kernel(x_ref, gw_ref, w1_ref, w3_ref, w2_ref, out_ref,
                wall_ref, scale_ref, xb_ref):
    e = pl.program_id(0)
    j = pl.program_id(1)

    @pl.when((e == 0) & (j == 0))
    def _gate():
        x = x_ref[...]
        xb_ref[...] = x.astype(jnp.bfloat16)
        logits = jnp.dot(x, gw_ref[...], preferred_element_type=jnp.float32)
        m0 = jnp.max(logits, axis=1, keepdims=True)
        cond0 = logits == m0
        masked = jnp.where(cond0, _MASK_VAL, logits)
        m1 = jnp.max(masked, axis=1, keepdims=True)
        cond1 = logits == m1
        pre = 1.0 / (1.0 + jnp.exp(m1 - m0))
        wall_ref[...] = (cond0.astype(jnp.float32) * pre
                         - cond1.astype(jnp.float32) * (pre - 1.0))
        out_ref[...] = jnp.zeros_like(out_ref)

    @pl.when(j == 0)
    def _scale():
        onehot = jax.lax.broadcasted_iota(jnp.int32, (1, E), 1) == e
        scale_ref[...] = jnp.sum(jnp.where(onehot, wall_ref[...], 0.0),
                                 axis=1, keepdims=True)

    xb = xb_ref[...]
    h1 = jnp.dot(xb, w1_ref[0].astype(jnp.bfloat16),
                 preferred_element_type=jnp.float32)
    h3 = jnp.dot(xb, w3_ref[0].astype(jnp.bfloat16),
                 preferred_element_type=jnp.float32)
    hidden = (h1 * jax.nn.sigmoid(h1)) * h3
    hidden = hidden * scale_ref[...]
    out_ref[...] += jnp.dot(hidden.astype(jnp.bfloat16),
                            w2_ref[0].astype(jnp.bfloat16),
                            preferred_element_type=jnp.float32)


@functools.partial(jax.jit, static_argnames=("interpret",))
def _moe(x, gate_w, w1, w3, w2, interpret=False):
    return pl.pallas_call(
        _moe_kernel,
        grid=(E, NF),
        in_specs=[
            pl.BlockSpec((B, H), lambda e, j: (0, 0)),
            pl.BlockSpec((H, E), lambda e, j: (0, 0)),
            pl.BlockSpec((1, H, BF), lambda e, j: (e, 0, j)),
            pl.BlockSpec((1, H, BF), lambda e, j: (e, 0, j)),
            pl.BlockSpec((1, BF, H), lambda e, j: (e, j, 0)),
        ],
        out_specs=pl.BlockSpec((B, H), lambda e, j: (0, 0)),
        out_shape=jax.ShapeDtypeStruct((B, H), jnp.float32),
        scratch_shapes=[
            pltpu.VMEM((B, E), jnp.float32),
            pltpu.VMEM((B, 1), jnp.float32),
            pltpu.VMEM((B, H), jnp.bfloat16),
        ],
        compiler_params=pltpu.CompilerParams(
            dimension_semantics=("arbitrary", "arbitrary"),
        ),
        interpret=interpret,
    )(x, gate_w, w1, w3, w2)


def kernel(inputs, gate_w, w1, w3, w2):
    x = inputs.reshape(B, H)
    out = _moe(x, gate_w, w1, w3, w2)
    return out.reshape(1, 1, B, H)
